# Initial kernel scaffold; baseline (speedup 1.0000x reference)
import jax
import jax.numpy as jnp
from jax.experimental import pallas as pl


def kernel(x, batch, gate_w, nn_w, nn_b):
    raise NotImplementedError("write your pallas kernel here")


# speedup vs baseline: 37.2975x; 37.2975x over previous
import jax
import jax.numpy as jnp
from jax.experimental import pallas as pl
from jax.experimental.pallas import tpu as pltpu

N_NODES = 50000
IN_CH = 256
NUM_HEADS = 4
OUT_CH = 256
OUT_PER_HEAD = OUT_CH // NUM_HEADS
NUM_GRAPHS = 64

BLK = 2000
NB = N_NODES // BLK


def _agg_kernel(x_ref, batch_ref, gate_w_ref, nn_w_ref, nn_b_ref, out_ref,
                m_ref, d_ref, pooled_ref):
    i = pl.program_id(0)

    @pl.when(i == 0)
    def _init():
        m_ref[...] = jnp.full((NUM_GRAPHS, NUM_HEADS), -jnp.inf, jnp.float32)
        d_ref[...] = jnp.zeros((NUM_GRAPHS, NUM_HEADS), jnp.float32)
        pooled_ref[...] = jnp.zeros((NUM_HEADS * NUM_GRAPHS, IN_CH), jnp.float32)

    xb = x_ref[...]
    bt = batch_ref[0]
    gateT = jax.lax.dot_general(
        gate_w_ref[...], xb, (((1,), (1,)), ((), ())),
        preferred_element_type=jnp.float32)
    gidx = jax.lax.broadcasted_iota(jnp.int32, (NUM_GRAPHS, BLK), 0)
    onehot = (bt == gidx)

    e_rows = []
    for h in range(NUM_HEADS):
        gh = gateT[h:h + 1, :]
        m_old = m_ref[:, h:h + 1]
        masked = jnp.where(onehot, gh, -jnp.inf)
        bm = jnp.max(masked, axis=1, keepdims=True)
        m_new = jnp.maximum(m_old, bm)
        eh = jnp.where(onehot, jnp.exp(gh - m_new), 0.0)
        bd = jnp.sum(eh, axis=1, keepdims=True)
        corr = jnp.where(m_old == -jnp.inf, 0.0,
                         jnp.exp(m_old - m_new))
        d_ref[:, h:h + 1] = d_ref[:, h:h + 1] * corr + bd
        m_ref[:, h:h + 1] = m_new
        sl = slice(h * NUM_GRAPHS, (h + 1) * NUM_GRAPHS)
        pooled_ref[sl, :] = pooled_ref[sl, :] * corr
        e_rows.append(eh)

    et = jnp.concatenate(e_rows, axis=0)
    pooled_ref[...] += jnp.dot(et, xb, preferred_element_type=jnp.float32)

    @pl.when(i == NB - 1)
    def _finalize():
        for h in range(NUM_HEADS):
            dh = d_ref[:, h:h + 1]
            safe = jnp.where(dh > 0.0, dh, 1.0)
            sl = slice(h * NUM_GRAPHS, (h + 1) * NUM_GRAPHS)
            ph = pooled_ref[sl, :] / safe
            oh = jax.lax.dot_general(
                ph, nn_w_ref[h], (((1,), (1,)), ((), ())),
                preferred_element_type=jnp.float32)
            oh = oh + jnp.where(dh > 0.0, 1.0, 0.0) * nn_b_ref[h:h + 1, :]
            out_ref[:, h * OUT_PER_HEAD:(h + 1) * OUT_PER_HEAD] = oh


def kernel(x, batch, gate_w, nn_w, nn_b):
    batch3d = batch.astype(jnp.int32).reshape(NB, 1, BLK)
    nn_b2 = nn_b.reshape(NUM_HEADS, OUT_PER_HEAD)
    return pl.pallas_call(
        _agg_kernel,
        grid=(NB,),
        in_specs=[
            pl.BlockSpec((BLK, IN_CH), lambda i: (i, 0)),
            pl.BlockSpec((1, 1, BLK), lambda i: (i, 0, 0)),
            pl.BlockSpec((NUM_HEADS, IN_CH), lambda i: (0, 0)),
            pl.BlockSpec((NUM_HEADS, OUT_PER_HEAD, IN_CH), lambda i: (0, 0, 0)),
            pl.BlockSpec((NUM_HEADS, OUT_PER_HEAD), lambda i: (0, 0)),
        ],
        out_specs=pl.BlockSpec((NUM_GRAPHS, OUT_CH), lambda i: (0, 0)),
        out_shape=jax.ShapeDtypeStruct((NUM_GRAPHS, OUT_CH), jnp.float32),
        scratch_shapes=[
            pltpu.VMEM((NUM_GRAPHS, NUM_HEADS), jnp.float32),
            pltpu.VMEM((NUM_GRAPHS, NUM_HEADS), jnp.float32),
            pltpu.VMEM((NUM_HEADS * NUM_GRAPHS, IN_CH), jnp.float32),
        ],
    )(x, batch3d, gate_w, nn_w, nn_b2)


# speedup vs baseline: 42.2095x; 1.1317x over previous
import jax
import jax.numpy as jnp
from jax.experimental import pallas as pl
from jax.experimental.pallas import tpu as pltpu

N_NODES = 50000
IN_CH = 256
NUM_HEADS = 4
OUT_CH = 256
OUT_PER_HEAD = OUT_CH // NUM_HEADS
NUM_GRAPHS = 64

BLK = 2000
NB = N_NODES // BLK


def _agg_kernel(x_ref, batch_ref, gate_w_ref, nn_w_ref, nn_b_ref, out_ref,
                m_ref, d_ref, pooled_ref):
    i = pl.program_id(0)

    @pl.when(i == 0)
    def _init():
        m_ref[...] = jnp.full((NUM_HEADS, 1), -jnp.inf, jnp.float32)
        d_ref[...] = jnp.zeros((NUM_GRAPHS, NUM_HEADS), jnp.float32)
        pooled_ref[...] = jnp.zeros((NUM_HEADS * NUM_GRAPHS, IN_CH), jnp.float32)

    xb = x_ref[...]
    bt = batch_ref[0]
    gateT = jax.lax.dot_general(
        gate_w_ref[...], xb, (((1,), (1,)), ((), ())),
        preferred_element_type=jnp.float32)
    gidx = jax.lax.broadcasted_iota(jnp.int32, (NUM_GRAPHS, BLK), 0)
    maskf = (bt == gidx).astype(jnp.float32)

    m_old = m_ref[...]
    bm = jnp.max(gateT, axis=1, keepdims=True)
    m_new = jnp.maximum(m_old, bm)
    corr = jnp.where(m_old == -jnp.inf, 0.0,
                     jnp.exp(m_old - m_new))
    m_ref[...] = m_new
    en = jnp.exp(gateT - m_new)

    bd = jax.lax.dot_general(
        maskf, en, (((1,), (1,)), ((), ())),
        preferred_element_type=jnp.float32)
    d_ref[...] = d_ref[...] * corr.reshape(1, NUM_HEADS) + bd

    e_rows = [maskf * en[h:h + 1, :] for h in range(NUM_HEADS)]
    et = jnp.concatenate(e_rows, axis=0)
    contrib = jnp.dot(et, xb, preferred_element_type=jnp.float32)
    for h in range(NUM_HEADS):
        sl = slice(h * NUM_GRAPHS, (h + 1) * NUM_GRAPHS)
        pooled_ref[sl, :] = pooled_ref[sl, :] * corr[h, 0] + contrib[sl, :]

    @pl.when(i == NB - 1)
    def _finalize():
        for h in range(NUM_HEADS):
            dh = d_ref[:, h:h + 1]
            safe = jnp.where(dh > 0.0, dh, 1.0)
            sl = slice(h * NUM_GRAPHS, (h + 1) * NUM_GRAPHS)
            ph = pooled_ref[sl, :] / safe
            oh = jax.lax.dot_general(
                ph, nn_w_ref[h], (((1,), (1,)), ((), ())),
                preferred_element_type=jnp.float32)
            oh = oh + jnp.where(dh > 0.0, 1.0, 0.0) * nn_b_ref[h:h + 1, :]
            out_ref[:, h * OUT_PER_HEAD:(h + 1) * OUT_PER_HEAD] = oh


def kernel(x, batch, gate_w, nn_w, nn_b):
    batch3d = batch.astype(jnp.int32).reshape(NB, 1, BLK)
    nn_b2 = nn_b.reshape(NUM_HEADS, OUT_PER_HEAD)
    return pl.pallas_call(
        _agg_kernel,
        grid=(NB,),
        in_specs=[
            pl.BlockSpec((BLK, IN_CH), lambda i: (i, 0)),
            pl.BlockSpec((1, 1, BLK), lambda i: (i, 0, 0)),
            pl.BlockSpec((NUM_HEADS, IN_CH), lambda i: (0, 0)),
            pl.BlockSpec((NUM_HEADS, OUT_PER_HEAD, IN_CH), lambda i: (0, 0, 0)),
            pl.BlockSpec((NUM_HEADS, OUT_PER_HEAD), lambda i: (0, 0)),
        ],
        out_specs=pl.BlockSpec((NUM_GRAPHS, OUT_CH), lambda i: (0, 0)),
        out_shape=jax.ShapeDtypeStruct((NUM_GRAPHS, OUT_CH), jnp.float32),
        scratch_shapes=[
            pltpu.VMEM((NUM_HEADS, 1), jnp.float32),
            pltpu.VMEM((NUM_GRAPHS, NUM_HEADS), jnp.float32),
            pltpu.VMEM((NUM_HEADS * NUM_GRAPHS, IN_CH), jnp.float32),
        ],
    )(x, batch3d, gate_w, nn_w, nn_b2)


# speedup vs baseline: 47.3346x; 1.1214x over previous
import jax
import jax.numpy as jnp
from jax.experimental import pallas as pl
from jax.experimental.pallas import tpu as pltpu

N_NODES = 50000
IN_CH = 256
NUM_HEADS = 4
OUT_CH = 256
OUT_PER_HEAD = OUT_CH // NUM_HEADS
NUM_GRAPHS = 64

BLK = 5000
NB = N_NODES // BLK


def _agg_kernel(x_ref, batch_ref, gate_w_ref, nn_w_ref, nn_b_ref, out_ref,
                m_ref, d_ref, pooled_ref):
    i = pl.program_id(0)

    @pl.when(i == 0)
    def _init():
        m_ref[...] = jnp.full((NUM_HEADS, 1), -jnp.inf, jnp.float32)
        d_ref[...] = jnp.zeros((NUM_GRAPHS, NUM_HEADS), jnp.float32)
        pooled_ref[...] = jnp.zeros((NUM_HEADS * NUM_GRAPHS, IN_CH), jnp.float32)

    xb = x_ref[...]
    bt = batch_ref[0]
    gateT = jax.lax.dot_general(
        gate_w_ref[...], xb, (((1,), (1,)), ((), ())),
        preferred_element_type=jnp.float32)
    gidx = jax.lax.broadcasted_iota(jnp.int32, (NUM_GRAPHS, BLK), 0)
    maskf = (bt == gidx).astype(jnp.float32)

    m_old = m_ref[...]
    bm = jnp.max(gateT, axis=1, keepdims=True)
    m_new = jnp.maximum(m_old, bm)
    corr = jnp.where(m_old == -jnp.inf, 0.0,
                     jnp.exp(m_old - m_new))
    m_ref[...] = m_new
    en = jnp.exp(gateT - m_new)

    en16 = en.astype(jnp.bfloat16)
    mask16 = maskf.astype(jnp.bfloat16)
    xb16 = xb.astype(jnp.bfloat16)

    bd = jax.lax.dot_general(
        mask16, en16, (((1,), (1,)), ((), ())),
        preferred_element_type=jnp.float32)
    d_ref[...] = d_ref[...] * corr.reshape(1, NUM_HEADS) + bd

    e_rows = [mask16 * en16[h:h + 1, :] for h in range(NUM_HEADS)]
    et = jnp.concatenate(e_rows, axis=0)
    contrib = jnp.dot(et, xb16, preferred_element_type=jnp.float32)
    for h in range(NUM_HEADS):
        sl = slice(h * NUM_GRAPHS, (h + 1) * NUM_GRAPHS)
        pooled_ref[sl, :] = pooled_ref[sl, :] * corr[h, 0] + contrib[sl, :]

    @pl.when(i == NB - 1)
    def _finalize():
        for h in range(NUM_HEADS):
            dh = d_ref[:, h:h + 1]
            safe = jnp.where(dh > 0.0, dh, 1.0)
            sl = slice(h * NUM_GRAPHS, (h + 1) * NUM_GRAPHS)
            ph = pooled_ref[sl, :] / safe
            oh = jax.lax.dot_general(
                ph, nn_w_ref[h], (((1,), (1,)), ((), ())),
                preferred_element_type=jnp.float32)
            oh = oh + jnp.where(dh > 0.0, 1.0, 0.0) * nn_b_ref[h:h + 1, :]
            out_ref[:, h * OUT_PER_HEAD:(h + 1) * OUT_PER_HEAD] = oh


def kernel(x, batch, gate_w, nn_w, nn_b):
    batch3d = batch.astype(jnp.int32).reshape(NB, 1, BLK)
    nn_b2 = nn_b.reshape(NUM_HEADS, OUT_PER_HEAD)
    return pl.pallas_call(
        _agg_kernel,
        grid=(NB,),
        in_specs=[
            pl.BlockSpec((BLK, IN_CH), lambda i: (i, 0)),
            pl.BlockSpec((1, 1, BLK), lambda i: (i, 0, 0)),
            pl.BlockSpec((NUM_HEADS, IN_CH), lambda i: (0, 0)),
            pl.BlockSpec((NUM_HEADS, OUT_PER_HEAD, IN_CH), lambda i: (0, 0, 0)),
            pl.BlockSpec((NUM_HEADS, OUT_PER_HEAD), lambda i: (0, 0)),
        ],
        out_specs=pl.BlockSpec((NUM_GRAPHS, OUT_CH), lambda i: (0, 0)),
        out_shape=jax.ShapeDtypeStruct((NUM_GRAPHS, OUT_CH), jnp.float32),
        scratch_shapes=[
            pltpu.VMEM((NUM_HEADS, 1), jnp.float32),
            pltpu.VMEM((NUM_GRAPHS, NUM_HEADS), jnp.float32),
            pltpu.VMEM((NUM_HEADS * NUM_GRAPHS, IN_CH), jnp.float32),
        ],
    )(x, batch3d, gate_w, nn_w, nn_b2)
